# Initial kernel scaffold; baseline (speedup 1.0000x reference)
import jax
import jax.numpy as jnp
from jax.experimental import pallas as pl


def kernel(nodes, walks, table):
    raise NotImplementedError("write your pallas kernel here")


# speedup vs baseline: 1.6780x; 1.6780x over previous
import jax
import jax.numpy as jnp
from jax import lax
from jax.experimental import pallas as pl
from jax.experimental.pallas import tpu as pltpu
from jax.experimental.pallas import tpu_sc as plsc

NUM_NODES = 1000000
EMBED = 16
BATCH = 16384
WALK_LEN = 50

NC = 2
NS = 16
NW = NC * NS
BPW = BATCH // NW
CB = 64
NCHUNK = BPW // CB
ROWS = CB * WALK_LEN
GATHER_W = 128
NGATHER = ROWS // GATHER_W
NODE_G = BPW // GATHER_W
WIDX_ROWS = BPW * WALK_LEN // GATHER_W


def _sc_body(table, nodes3d, walks3d, out_emb, partials,
             nidx, nrows, widx, wrows, pacc, sem):
    wid = lax.axis_index("s") * NC + lax.axis_index("c")
    base = wid * BPW

    pltpu.sync_copy(nodes3d.at[wid], nidx)
    pltpu.sync_copy(walks3d.at[wid], widx)

    nh = [pltpu.async_copy(table.at[nidx.at[j]],
                           nrows.at[pl.ds(j * GATHER_W, GATHER_W)], sem)
          for j in range(NODE_G)]
    for h in nh:
        h.wait()
    pltpu.sync_copy(nrows, out_emb.at[pl.ds(base, BPW)])

    def chunk_body(c, acc):
        hs = [pltpu.async_copy(table.at[widx.at[c * NGATHER + j]],
                               wrows.at[pl.ds(j * GATHER_W, GATHER_W)], sem)
              for j in range(NGATHER)]
        for h in hs:
            h.wait()

        def b_body(b, acc):
            m0 = b * WALK_LEN
            wsum = wrows[m0, :]
            for l in range(1, WALK_LEN):
                wsum = wsum + wrows[m0 + l, :]
            return acc + nrows[c * CB + b, :] * wsum

        return lax.fori_loop(0, CB, b_body, acc)

    acc = lax.fori_loop(0, NCHUNK, chunk_body,
                        jnp.zeros((EMBED,), jnp.float32))
    pacc[0, :] = acc
    pltpu.sync_copy(pacc, partials.at[wid])


@jax.jit
def _sc_call(table, nodes3d, walks3d):
    mesh = plsc.VectorSubcoreMesh(core_axis_name="c", subcore_axis_name="s",
                                  num_cores=NC, num_subcores=NS)
    return pl.kernel(
        _sc_body,
        out_type=(
            jax.ShapeDtypeStruct((BATCH, EMBED), jnp.float32),
            jax.ShapeDtypeStruct((NW, 1, EMBED), jnp.float32),
        ),
        mesh=mesh,
        compiler_params=pltpu.CompilerParams(use_tc_tiling_on_sc=False),
        scratch_types=(
            pltpu.VMEM((NODE_G, GATHER_W), jnp.int32),
            pltpu.VMEM((BPW, EMBED), jnp.float32),
            pltpu.VMEM((WIDX_ROWS, GATHER_W), jnp.int32),
            pltpu.VMEM((ROWS, EMBED), jnp.float32),
            pltpu.VMEM((1, EMBED), jnp.float32),
            pltpu.SemaphoreType.DMA,
        ),
    )(table, nodes3d, walks3d)


def kernel(nodes, walks, table):
    nodes3d = nodes.astype(jnp.int32).reshape(NW, NODE_G, GATHER_W)
    walks3d = walks.astype(jnp.int32).reshape(NW, WIDX_ROWS, GATHER_W)
    out_emb, partials = _sc_call(table, nodes3d, walks3d)
    return (out_emb, jnp.sum(partials))
